# baseline (speedup 1.0000x reference)
import functools

import jax
import jax.numpy as jnp
from jax import lax
from jax.experimental import pallas as pl
from jax.experimental.pallas import tpu as pltpu
from jax.experimental.pallas import tpu_sc as plsc

N_NODES = 10000
N_PAD = 10240
N_EDGES = 320000
F = 128
NC = 2
NS = 16
NW = NC * NS
CH = 64
E_PER_TILE = 10240
N_CHUNKS = E_PER_TILE // CH
RPS = N_PAD // NS


NB = 2


@functools.lru_cache(maxsize=None)
def _sc_segsum_kernel(D):
    mesh = plsc.VectorSubcoreMesh(core_axis_name="c", subcore_axis_name="s")

    @functools.partial(
        pl.kernel,
        mesh=mesh,
        out_type=jax.ShapeDtypeStruct((NC * N_PAD, D), jnp.float32),
        scratch_types=[
            pltpu.VMEM((E_PER_TILE,), jnp.int32),
            pltpu.VMEM((N_CHUNKS, CH), jnp.int32),
        ] + [pltpu.VMEM((CH, D), jnp.float32) for _ in range(NB)]
          + [pltpu.VMEM_SHARED((N_PAD, D), jnp.float32)]
          + [pltpu.SemaphoreType.DMA for _ in range(NB)],
    )
    def k(table_hbm, src_hbm, dst_hbm, zeros_hbm, out_hbm, sidx, didx,
          *rest):
        rows = rest[:NB]
        acc = rest[NB]
        sems = rest[NB + 1:]
        cid = lax.axis_index("c")
        sid = lax.axis_index("s")
        wid = sid * NC + cid
        pltpu.sync_copy(src_hbm.at[pl.ds(wid * E_PER_TILE, E_PER_TILE)], sidx)
        pltpu.sync_copy(dst_hbm.at[wid], didx)
        pltpu.sync_copy(zeros_hbm.at[pl.ds(sid * RPS, RPS)],
                        acc.at[pl.ds(sid * RPS, RPS)])
        plsc.subcore_barrier()

        for b in range(NB):
            pltpu.async_copy(table_hbm.at[sidx.at[pl.ds(b * CH, CH)]],
                             rows[b], sems[b])

        def body(g, carry):
            for b in range(NB):
                i = g * NB + b
                pltpu.make_async_copy(
                    table_hbm.at[sidx.at[pl.ds(i * CH, CH)]], rows[b],
                    sems[b]).wait()
                pltpu.sync_copy(rows[b], acc.at[didx.at[i]], add=True)

                @pl.when(i + NB < N_CHUNKS)
                def _():
                    pltpu.async_copy(
                        table_hbm.at[sidx.at[pl.ds((i + NB) * CH, CH)]],
                        rows[b], sems[b])
            return carry

        lax.fori_loop(0, N_CHUNKS // NB, body, 0)
        plsc.subcore_barrier()
        pltpu.sync_copy(acc.at[pl.ds(sid * RPS, RPS)],
                        out_hbm.at[pl.ds(cid * N_PAD + sid * RPS, RPS)])

    return k


def _segsum(table, src, dst, zeros):
    return _sc_segsum_kernel(table.shape[1])(table, src, dst, zeros)


def _pad_tiles(v):
    v2 = v.reshape(NW, -1)
    pad = jnp.full((NW, E_PER_TILE - v2.shape[1]), N_PAD - 1, v.dtype)
    return jnp.concatenate([v2, pad], axis=1)


def _mm(x, w):
    return jnp.dot(x, w, preferred_element_type=jnp.float32,
                   precision=lax.Precision.HIGHEST)


def _tk1_body(x_ref, w_ref, degp_ref, m_ref, h_ref, hp_ref, deg_ref, a_ref):
    m = m_ref[...]
    degs = degp_ref[:N_PAD, :] + degp_ref[N_PAD:, :]
    deg = m * jnp.sum(degs, axis=1) + 1.0
    a = m * lax.rsqrt(deg)
    h = _mm(x_ref[...], w_ref[...])
    h_ref[...] = h
    hp_ref[...] = a[:, None] * h
    deg_ref[...] = deg
    a_ref[...] = a


@jax.jit
def _tk1(x, w, degp, m):
    return pl.pallas_call(
        _tk1_body,
        out_shape=(
            jax.ShapeDtypeStruct((N_PAD, F), jnp.float32),
            jax.ShapeDtypeStruct((N_PAD, F), jnp.float32),
            jax.ShapeDtypeStruct((N_PAD,), jnp.float32),
            jax.ShapeDtypeStruct((N_PAD,), jnp.float32),
        ),
    )(x, w, degp, m)


def _tk2_body(h_ref, aggp_ref, a_ref, deg_ref, m_ref, b_ref, out_ref, tb_ref):
    aggs = aggp_ref[:N_PAD, :] + aggp_ref[N_PAD:, :]
    deg = deg_ref[...]
    out = jax.nn.relu(a_ref[...][:, None] * aggs
                      + h_ref[...] * (1.0 / deg)[:, None] + b_ref[...][None, :])
    out_ref[...] = out
    tb_ref[...] = m_ref[...][:, None] * out


@jax.jit
def _tk2(h, aggp, a, deg, m, b):
    return pl.pallas_call(
        _tk2_body,
        out_shape=(
            jax.ShapeDtypeStruct((N_PAD, F), jnp.float32),
            jax.ShapeDtypeStruct((N_PAD, F), jnp.float32),
        ),
    )(h, aggp, a, deg, m, b)


def _kth_largest(score, k):
    def body(i, carry):
        lo, hi = carry
        mid = lo + (hi - lo) // 2
        t = lax.bitcast_convert_type(mid, jnp.float32)
        cnt = jnp.sum((score >= t).astype(jnp.int32))
        take = cnt >= k
        return jnp.where(take, mid, lo), jnp.where(take, hi, mid)

    lo, _ = lax.fori_loop(0, 31, body, (jnp.int32(0), jnp.int32(0x7F800000)))
    return lax.bitcast_convert_type(lo, jnp.float32)


def _tk3_body(k, out_ref, neighp_ref, deg_ref, m_ref, xn_ref, mn_ref,
              m16_ref, xr_ref):
    out = out_ref[...]
    m = m_ref[...]
    deg = deg_ref[...]
    neighs = neighp_ref[:N_PAD, :] + neighp_ref[N_PAD:, :]
    deg2 = jnp.maximum(deg - 1.0, 1.0)
    neigh = m[:, None] * neighs / deg2[:, None]
    score = jnp.sum(jnp.abs(out - neigh), axis=1)
    score = jnp.where(m > 0, score, -1.0)
    t = _kth_largest(score, k)
    sel = (score >= t).astype(jnp.float32)
    xn = out * jnp.tanh(score)[:, None]
    xn_ref[...] = xn
    mn_ref[...] = sel
    lane = lax.broadcasted_iota(jnp.int32, (N_PAD, F), 1)
    m16_ref[...] = jnp.where(lane == 0, sel[:, None], 0.0)
    xs = xn * sel[:, None]
    ssum = jnp.sum(xs, axis=0) * (1.0 / k)
    smax = jnp.max(jnp.where(sel[:, None] > 0, xn, -jnp.inf), axis=0)
    xr_ref[...] = jnp.concatenate([smax, ssum]).reshape(1, 2 * F)


@functools.partial(jax.jit, static_argnums=4)
def _tk3(out, neighp, deg, m, k):
    return pl.pallas_call(
        functools.partial(_tk3_body, k),
        out_shape=(
            jax.ShapeDtypeStruct((N_PAD, F), jnp.float32),
            jax.ShapeDtypeStruct((N_PAD,), jnp.float32),
            jax.ShapeDtypeStruct((N_PAD, F), jnp.float32),
            jax.ShapeDtypeStruct((1, 2 * F), jnp.float32),
        ),
    )(out, neighp, deg, m)


def _tkf_body(k, h_ref, aggp_ref, a_ref, deg_ref, m_ref, b_ref, x1_ref,
              x2_ref, y_ref):
    aggs = aggp_ref[:N_PAD, :] + aggp_ref[N_PAD:, :]
    deg = deg_ref[...]
    out = jax.nn.relu(a_ref[...][:, None] * aggs
                      + h_ref[...] * (1.0 / deg)[:, None] + b_ref[...][None, :])
    m = m_ref[...]
    ssum = jnp.sum(out * m[:, None], axis=0) * (1.0 / k)
    smax = jnp.max(jnp.where(m[:, None] > 0, out, -jnp.inf), axis=0)
    x3 = jnp.concatenate([smax, ssum]).reshape(1, 2 * F)
    y_ref[...] = (jax.nn.relu(x1_ref[...]) + jax.nn.relu(x2_ref[...])
                  + jax.nn.relu(x3))


@functools.partial(jax.jit, static_argnums=8)
def _tkf(h, aggp, a, deg, m, b, x1, x2, k):
    return pl.pallas_call(
        functools.partial(_tkf_body, k),
        out_shape=jax.ShapeDtypeStruct((1, 2 * F), jnp.float32),
    )(h, aggp, a, deg, m, b, x1, x2)


def kernel(x, edge_index, batch, W1, b1, W2, b2, W3, b3):
    src = _pad_tiles(edge_index[0].astype(jnp.int32)).reshape(-1)
    dst = _pad_tiles(edge_index[1].astype(jnp.int32)).reshape(NW, N_CHUNKS, CH)
    xp = jnp.zeros((N_PAD, F), jnp.float32).at[:N_NODES].set(x)
    node = jnp.arange(N_PAD, dtype=jnp.int32)
    m = (node < N_NODES).astype(jnp.float32)
    m16 = jnp.zeros((N_PAD, F), jnp.float32).at[:, 0].set(m)
    zeros_f = jnp.zeros((N_PAD, F), jnp.float32)

    ks = [5000, 2500, 2500]
    readouts = []
    x_cur = xp
    for layer, (W, b) in enumerate([(W1, b1), (W2, b2), (W3, b3)]):
        degp = _segsum(m16, src, dst, zeros_f)
        h, hp, deg, a = _tk1(x_cur, W, degp, m)
        aggp = _segsum(hp, src, dst, zeros_f)
        if layer < 2:
            out, tb = _tk2(h, aggp, a, deg, m, b)
            neighp = _segsum(tb, src, dst, zeros_f)
            x_cur, m, m16, xr = _tk3(out, neighp, deg, m, ks[layer])
            readouts.append(xr)
        else:
            x1, x2 = readouts
            return _tkf(h, aggp, a, deg, m, b, x1, x2, ks[layer])


# speedup vs baseline: 1.9095x; 1.9095x over previous
import functools

import jax
import jax.numpy as jnp
from jax import lax
from jax.experimental import pallas as pl
from jax.experimental.pallas import tpu as pltpu
from jax.experimental.pallas import tpu_sc as plsc

N_NODES = 10000
N_PAD = 10240
N_EDGES = 320000
F = 128
NC = 2
NS = 16
NW = NC * NS
CH = 80
E_PER_TILE = N_EDGES // NW
N_CHUNKS = E_PER_TILE // CH
RPS = N_PAD // NS


@functools.lru_cache(maxsize=None)
def _sc_segsum_kernel(D):
    mesh = plsc.VectorSubcoreMesh(core_axis_name="c", subcore_axis_name="s")

    @functools.partial(
        pl.kernel,
        mesh=mesh,
        out_type=jax.ShapeDtypeStruct((NC * N_PAD, D), jnp.float32),
        scratch_types=[
            pltpu.VMEM((E_PER_TILE,), jnp.int32),
            pltpu.VMEM((N_CHUNKS, CH), jnp.int32),
            pltpu.VMEM((CH, D), jnp.float32),
            pltpu.VMEM_SHARED((N_PAD, D), jnp.float32),
            pltpu.SemaphoreType.DMA,
        ],
    )
    def k(table_hbm, src_hbm, dst_hbm, zeros_hbm, out_hbm, sidx, didx,
          rows, acc, sem):
        cid = lax.axis_index("c")
        sid = lax.axis_index("s")
        wid = sid * NC + cid
        pltpu.sync_copy(src_hbm.at[pl.ds(wid * E_PER_TILE, E_PER_TILE)], sidx)
        pltpu.sync_copy(dst_hbm.at[wid], didx)
        pltpu.sync_copy(zeros_hbm.at[pl.ds(sid * RPS, RPS)],
                        acc.at[pl.ds(sid * RPS, RPS)])
        plsc.subcore_barrier()

        def body(i, carry):
            pltpu.async_copy(table_hbm.at[sidx.at[pl.ds(i * CH, CH)]],
                             rows, sem).wait()
            pltpu.sync_copy(rows, acc.at[didx.at[i]], add=True)
            return carry

        lax.fori_loop(0, N_CHUNKS, body, 0)
        plsc.subcore_barrier()
        pltpu.sync_copy(acc.at[pl.ds(sid * RPS, RPS)],
                        out_hbm.at[pl.ds(cid * N_PAD + sid * RPS, RPS)])

    return k


def _segsum(table, src, dst, zeros):
    return _sc_segsum_kernel(table.shape[1])(table, src, dst, zeros)


def _pad_tiles(v):
    v2 = v.reshape(NW, -1)
    pad = jnp.full((NW, E_PER_TILE - v2.shape[1]), N_PAD - 1, v.dtype)
    return jnp.concatenate([v2, pad], axis=1)


def _mm(x, w):
    return jnp.dot(x, w, preferred_element_type=jnp.float32,
                   precision=lax.Precision.HIGHEST)


def _tk1_body(x_ref, w_ref, degp_ref, m_ref, h_ref, hp_ref, deg_ref, a_ref):
    m = m_ref[...]
    degs = degp_ref[:N_PAD, :] + degp_ref[N_PAD:, :]
    deg = m * jnp.sum(degs, axis=1) + 1.0
    a = m * lax.rsqrt(deg)
    h = _mm(x_ref[...], w_ref[...])
    h_ref[...] = h
    hp_ref[...] = a[:, None] * h
    deg_ref[...] = deg
    a_ref[...] = a


@jax.jit
def _tk1(x, w, degp, m):
    return pl.pallas_call(
        _tk1_body,
        out_shape=(
            jax.ShapeDtypeStruct((N_PAD, F), jnp.float32),
            jax.ShapeDtypeStruct((N_PAD, F), jnp.float32),
            jax.ShapeDtypeStruct((N_PAD,), jnp.float32),
            jax.ShapeDtypeStruct((N_PAD,), jnp.float32),
        ),
    )(x, w, degp, m)


def _tk2_body(h_ref, aggp_ref, a_ref, deg_ref, m_ref, b_ref, out_ref, tb_ref):
    aggs = aggp_ref[:N_PAD, :] + aggp_ref[N_PAD:, :]
    deg = deg_ref[...]
    out = jax.nn.relu(a_ref[...][:, None] * aggs
                      + h_ref[...] * (1.0 / deg)[:, None] + b_ref[...][None, :])
    out_ref[...] = out
    tb_ref[...] = m_ref[...][:, None] * out


@jax.jit
def _tk2(h, aggp, a, deg, m, b):
    return pl.pallas_call(
        _tk2_body,
        out_shape=(
            jax.ShapeDtypeStruct((N_PAD, F), jnp.float32),
            jax.ShapeDtypeStruct((N_PAD, F), jnp.float32),
        ),
    )(h, aggp, a, deg, m, b)


def _kth_largest(score, k):
    def body(i, carry):
        lo, hi = carry
        mid = lo + (hi - lo) // 2
        t = lax.bitcast_convert_type(mid, jnp.float32)
        cnt = jnp.sum((score >= t).astype(jnp.int32))
        take = cnt >= k
        return jnp.where(take, mid, lo), jnp.where(take, hi, mid)

    lo, _ = lax.fori_loop(0, 31, body, (jnp.int32(0), jnp.int32(0x7F800000)))
    return lax.bitcast_convert_type(lo, jnp.float32)


def _tk3_body(k, out_ref, neighp_ref, deg_ref, m_ref, xn_ref, mn_ref,
              m16_ref, xr_ref):
    out = out_ref[...]
    m = m_ref[...]
    deg = deg_ref[...]
    neighs = neighp_ref[:N_PAD, :] + neighp_ref[N_PAD:, :]
    deg2 = jnp.maximum(deg - 1.0, 1.0)
    neigh = m[:, None] * neighs / deg2[:, None]
    score = jnp.sum(jnp.abs(out - neigh), axis=1)
    score = jnp.where(m > 0, score, -1.0)
    t = _kth_largest(score, k)
    sel = (score >= t).astype(jnp.float32)
    xn = out * jnp.tanh(score)[:, None]
    xn_ref[...] = xn
    mn_ref[...] = sel
    lane = lax.broadcasted_iota(jnp.int32, (N_PAD, F), 1)
    m16_ref[...] = jnp.where(lane == 0, sel[:, None], 0.0)
    xs = xn * sel[:, None]
    ssum = jnp.sum(xs, axis=0) * (1.0 / k)
    smax = jnp.max(jnp.where(sel[:, None] > 0, xn, -jnp.inf), axis=0)
    xr_ref[...] = jnp.concatenate([smax, ssum]).reshape(1, 2 * F)


@functools.partial(jax.jit, static_argnums=4)
def _tk3(out, neighp, deg, m, k):
    return pl.pallas_call(
        functools.partial(_tk3_body, k),
        out_shape=(
            jax.ShapeDtypeStruct((N_PAD, F), jnp.float32),
            jax.ShapeDtypeStruct((N_PAD,), jnp.float32),
            jax.ShapeDtypeStruct((N_PAD, F), jnp.float32),
            jax.ShapeDtypeStruct((1, 2 * F), jnp.float32),
        ),
    )(out, neighp, deg, m)


def _tkf_body(k, h_ref, aggp_ref, a_ref, deg_ref, m_ref, b_ref, x1_ref,
              x2_ref, y_ref):
    aggs = aggp_ref[:N_PAD, :] + aggp_ref[N_PAD:, :]
    deg = deg_ref[...]
    out = jax.nn.relu(a_ref[...][:, None] * aggs
                      + h_ref[...] * (1.0 / deg)[:, None] + b_ref[...][None, :])
    m = m_ref[...]
    ssum = jnp.sum(out * m[:, None], axis=0) * (1.0 / k)
    smax = jnp.max(jnp.where(m[:, None] > 0, out, -jnp.inf), axis=0)
    x3 = jnp.concatenate([smax, ssum]).reshape(1, 2 * F)
    y_ref[...] = (jax.nn.relu(x1_ref[...]) + jax.nn.relu(x2_ref[...])
                  + jax.nn.relu(x3))


@functools.partial(jax.jit, static_argnums=8)
def _tkf(h, aggp, a, deg, m, b, x1, x2, k):
    return pl.pallas_call(
        functools.partial(_tkf_body, k),
        out_shape=jax.ShapeDtypeStruct((1, 2 * F), jnp.float32),
    )(h, aggp, a, deg, m, b, x1, x2)


def kernel(x, edge_index, batch, W1, b1, W2, b2, W3, b3):
    src = _pad_tiles(edge_index[0].astype(jnp.int32)).reshape(-1)
    dst = _pad_tiles(edge_index[1].astype(jnp.int32)).reshape(NW, N_CHUNKS, CH)
    xp = jnp.zeros((N_PAD, F), jnp.float32).at[:N_NODES].set(x)
    node = jnp.arange(N_PAD, dtype=jnp.int32)
    m = (node < N_NODES).astype(jnp.float32)
    m16 = jnp.zeros((N_PAD, F), jnp.float32).at[:, 0].set(m)
    zeros_f = jnp.zeros((N_PAD, F), jnp.float32)

    ks = [5000, 2500, 2500]
    readouts = []
    x_cur = xp
    for layer, (W, b) in enumerate([(W1, b1), (W2, b2), (W3, b3)]):
        degp = _segsum(m16, src, dst, zeros_f)
        h, hp, deg, a = _tk1(x_cur, W, degp, m)
        aggp = _segsum(hp, src, dst, zeros_f)
        if layer < 2:
            out, tb = _tk2(h, aggp, a, deg, m, b)
            neighp = _segsum(tb, src, dst, zeros_f)
            x_cur, m, m16, xr = _tk3(out, neighp, deg, m, ks[layer])
            readouts.append(xr)
        else:
            x1, x2 = readouts
            return _tkf(h, aggp, a, deg, m, b, x1, x2, ks[layer])


# speedup vs baseline: 3.1024x; 1.6247x over previous
import functools

import jax
import jax.numpy as jnp
from jax import lax
from jax.experimental import pallas as pl
from jax.experimental.pallas import tpu as pltpu
from jax.experimental.pallas import tpu_sc as plsc

N_NODES = 10000
N_PAD = 10240
N_EDGES = 320000
F = 128
NC = 2
NS = 16
NW = NC * NS
CH = 80
E_PER_TILE = N_EDGES // NW
N_CHUNKS = E_PER_TILE // CH
RPS = N_PAD // NS


@functools.lru_cache(maxsize=None)
def _sc_segsum_kernel(D):
    mesh = plsc.VectorSubcoreMesh(core_axis_name="c", subcore_axis_name="s")

    @functools.partial(
        pl.kernel,
        mesh=mesh,
        out_type=jax.ShapeDtypeStruct((NC * N_PAD, D), jnp.float32),
        scratch_types=[
            pltpu.VMEM((E_PER_TILE,), jnp.int32),
            pltpu.VMEM((N_CHUNKS, CH), jnp.int32),
            pltpu.VMEM((CH, D), jnp.float32),
            pltpu.VMEM((CH, D), jnp.float32),
            pltpu.VMEM_SHARED((N_PAD, D), jnp.float32),
            pltpu.SemaphoreType.DMA,
            pltpu.SemaphoreType.DMA,
        ],
    )
    def k(table_hbm, src_hbm, dst_hbm, zeros_hbm, out_hbm, sidx, didx,
          r0, r1, acc, s0, s1):
        cid = lax.axis_index("c")
        sid = lax.axis_index("s")
        wid = sid * NC + cid
        pltpu.sync_copy(src_hbm.at[pl.ds(wid * E_PER_TILE, E_PER_TILE)], sidx)
        pltpu.sync_copy(dst_hbm.at[wid], didx)
        pltpu.sync_copy(zeros_hbm.at[pl.ds(sid * RPS, RPS)],
                        acc.at[pl.ds(sid * RPS, RPS)])
        plsc.subcore_barrier()

        def gat(i, r, s):
            pltpu.async_copy(table_hbm.at[sidx.at[pl.ds(i * CH, CH)]], r, s)

        def gwait(i, r, s):
            pltpu.make_async_copy(
                table_hbm.at[sidx.at[pl.ds(i * CH, CH)]], r, s).wait()

        gat(0, r0, s0)
        gat(1, r1, s1)

        def body(g, carry):
            i = g * 2
            gwait(i, r0, s0)
            pltpu.sync_copy(r0, acc.at[didx.at[i]], add=True)

            @pl.when(i + 2 < N_CHUNKS)
            def _():
                gat(i + 2, r0, s0)

            gwait(i + 1, r1, s1)
            pltpu.sync_copy(r1, acc.at[didx.at[i + 1]], add=True)

            @pl.when(i + 3 < N_CHUNKS)
            def _():
                gat(i + 3, r1, s1)

            return carry

        lax.fori_loop(0, N_CHUNKS // 2, body, 0)
        gwait(N_CHUNKS - 1, r0, s0)
        pltpu.sync_copy(r0, acc.at[didx.at[N_CHUNKS - 1]], add=True)
        plsc.subcore_barrier()
        pltpu.sync_copy(acc.at[pl.ds(sid * RPS, RPS)],
                        out_hbm.at[pl.ds(cid * N_PAD + sid * RPS, RPS)])

    return k


def _segsum(table, src, dst, zeros):
    return _sc_segsum_kernel(table.shape[1])(table, src, dst, zeros)


def _pad_tiles(v):
    v2 = v.reshape(NW, -1)
    pad = jnp.full((NW, E_PER_TILE - v2.shape[1]), N_PAD - 1, v.dtype)
    return jnp.concatenate([v2, pad], axis=1)


def _mm(x, w):
    return jnp.dot(x, w, preferred_element_type=jnp.float32,
                   precision=lax.Precision.HIGHEST)


def _tk1_body(x_ref, w_ref, degp_ref, m_ref, h_ref, hp_ref, deg_ref, a_ref):
    m = m_ref[...]
    degs = degp_ref[:N_PAD, :] + degp_ref[N_PAD:, :]
    deg = m * jnp.sum(degs, axis=1) + 1.0
    a = m * lax.rsqrt(deg)
    h = _mm(x_ref[...], w_ref[...])
    h_ref[...] = h
    hp_ref[...] = a[:, None] * h
    deg_ref[...] = deg
    a_ref[...] = a


@jax.jit
def _tk1(x, w, degp, m):
    return pl.pallas_call(
        _tk1_body,
        out_shape=(
            jax.ShapeDtypeStruct((N_PAD, F), jnp.float32),
            jax.ShapeDtypeStruct((N_PAD, F), jnp.float32),
            jax.ShapeDtypeStruct((N_PAD,), jnp.float32),
            jax.ShapeDtypeStruct((N_PAD,), jnp.float32),
        ),
    )(x, w, degp, m)


def _tk2_body(h_ref, aggp_ref, a_ref, deg_ref, m_ref, b_ref, out_ref, tb_ref):
    aggs = aggp_ref[:N_PAD, :] + aggp_ref[N_PAD:, :]
    deg = deg_ref[...]
    out = jax.nn.relu(a_ref[...][:, None] * aggs
                      + h_ref[...] * (1.0 / deg)[:, None] + b_ref[...][None, :])
    out_ref[...] = out
    tb_ref[...] = m_ref[...][:, None] * out


@jax.jit
def _tk2(h, aggp, a, deg, m, b):
    return pl.pallas_call(
        _tk2_body,
        out_shape=(
            jax.ShapeDtypeStruct((N_PAD, F), jnp.float32),
            jax.ShapeDtypeStruct((N_PAD, F), jnp.float32),
        ),
    )(h, aggp, a, deg, m, b)


def _kth_largest(score, k):
    def body(i, carry):
        lo, hi = carry
        mid = lo + (hi - lo) // 2
        t = lax.bitcast_convert_type(mid, jnp.float32)
        cnt = jnp.sum((score >= t).astype(jnp.int32))
        take = cnt >= k
        return jnp.where(take, mid, lo), jnp.where(take, hi, mid)

    lo, _ = lax.fori_loop(0, 31, body, (jnp.int32(0), jnp.int32(0x7F800000)))
    return lax.bitcast_convert_type(lo, jnp.float32)


def _tk3_body(k, out_ref, neighp_ref, deg_ref, m_ref, xn_ref, mn_ref,
              m16_ref, xr_ref):
    out = out_ref[...]
    m = m_ref[...]
    deg = deg_ref[...]
    neighs = neighp_ref[:N_PAD, :] + neighp_ref[N_PAD:, :]
    deg2 = jnp.maximum(deg - 1.0, 1.0)
    neigh = m[:, None] * neighs / deg2[:, None]
    score = jnp.sum(jnp.abs(out - neigh), axis=1)
    score = jnp.where(m > 0, score, -1.0)
    t = _kth_largest(score, k)
    sel = (score >= t).astype(jnp.float32)
    xn = out * jnp.tanh(score)[:, None]
    xn_ref[...] = xn
    mn_ref[...] = sel
    lane = lax.broadcasted_iota(jnp.int32, (N_PAD, F), 1)
    m16_ref[...] = jnp.where(lane == 0, sel[:, None], 0.0)
    xs = xn * sel[:, None]
    ssum = jnp.sum(xs, axis=0) * (1.0 / k)
    smax = jnp.max(jnp.where(sel[:, None] > 0, xn, -jnp.inf), axis=0)
    xr_ref[...] = jnp.concatenate([smax, ssum]).reshape(1, 2 * F)


@functools.partial(jax.jit, static_argnums=4)
def _tk3(out, neighp, deg, m, k):
    return pl.pallas_call(
        functools.partial(_tk3_body, k),
        out_shape=(
            jax.ShapeDtypeStruct((N_PAD, F), jnp.float32),
            jax.ShapeDtypeStruct((N_PAD,), jnp.float32),
            jax.ShapeDtypeStruct((N_PAD, F), jnp.float32),
            jax.ShapeDtypeStruct((1, 2 * F), jnp.float32),
        ),
    )(out, neighp, deg, m)


def _tkf_body(k, h_ref, aggp_ref, a_ref, deg_ref, m_ref, b_ref, x1_ref,
              x2_ref, y_ref):
    aggs = aggp_ref[:N_PAD, :] + aggp_ref[N_PAD:, :]
    deg = deg_ref[...]
    out = jax.nn.relu(a_ref[...][:, None] * aggs
                      + h_ref[...] * (1.0 / deg)[:, None] + b_ref[...][None, :])
    m = m_ref[...]
    ssum = jnp.sum(out * m[:, None], axis=0) * (1.0 / k)
    smax = jnp.max(jnp.where(m[:, None] > 0, out, -jnp.inf), axis=0)
    x3 = jnp.concatenate([smax, ssum]).reshape(1, 2 * F)
    y_ref[...] = (jax.nn.relu(x1_ref[...]) + jax.nn.relu(x2_ref[...])
                  + jax.nn.relu(x3))


@functools.partial(jax.jit, static_argnums=8)
def _tkf(h, aggp, a, deg, m, b, x1, x2, k):
    return pl.pallas_call(
        functools.partial(_tkf_body, k),
        out_shape=jax.ShapeDtypeStruct((1, 2 * F), jnp.float32),
    )(h, aggp, a, deg, m, b, x1, x2)


def kernel(x, edge_index, batch, W1, b1, W2, b2, W3, b3):
    src = _pad_tiles(edge_index[0].astype(jnp.int32)).reshape(-1)
    dst = _pad_tiles(edge_index[1].astype(jnp.int32)).reshape(NW, N_CHUNKS, CH)
    xp = jnp.zeros((N_PAD, F), jnp.float32).at[:N_NODES].set(x)
    node = jnp.arange(N_PAD, dtype=jnp.int32)
    m = (node < N_NODES).astype(jnp.float32)
    m16 = jnp.zeros((N_PAD, F), jnp.float32).at[:, 0].set(m)
    zeros_f = jnp.zeros((N_PAD, F), jnp.float32)

    ks = [5000, 2500, 2500]
    readouts = []
    x_cur = xp
    for layer, (W, b) in enumerate([(W1, b1), (W2, b2), (W3, b3)]):
        degp = _segsum(m16, src, dst, zeros_f)
        h, hp, deg, a = _tk1(x_cur, W, degp, m)
        aggp = _segsum(hp, src, dst, zeros_f)
        if layer < 2:
            out, tb = _tk2(h, aggp, a, deg, m, b)
            neighp = _segsum(tb, src, dst, zeros_f)
            x_cur, m, m16, xr = _tk3(out, neighp, deg, m, ks[layer])
            readouts.append(xr)
        else:
            x1, x2 = readouts
            return _tkf(h, aggp, a, deg, m, b, x1, x2, ks[layer])
